# baseline (speedup 1.0000x reference)
import functools

import jax
import jax.numpy as jnp
from jax.experimental import pallas as pl
from jax.experimental.pallas import tpu as pltpu

NUM_EXPERTS = 8
TOP_K = 2
SL = 2048
HS = 768
FF = 3072
M = SL * TOP_K

TM = 256
NT = M // TM
L = NT + NUM_EXPERTS - 1
FFT = 768
F = FF // FFT

TMS = 256
NTS = SL // TMS


def _grouped_body(meta_ref, x_ref, w1_ref, w2_ref, scale_ref, y_ref):
    l = pl.program_id(0)
    f = pl.program_id(1)
    first = meta_ref[4 * L + l]

    @pl.when((f == 0) & (first == 1))
    def _():
        y_ref[...] = jnp.zeros_like(y_ref)

    h = jax.nn.gelu(jnp.dot(x_ref[...], w1_ref[0],
                            preferred_element_type=jnp.float32))
    m = meta_ref[L + l]
    start = meta_ref[2 * L + l]
    end = meta_ref[3 * L + l]
    rows = m * TM + jax.lax.broadcasted_iota(jnp.int32, (TM, 1), 0)
    scale = scale_ref[0, 0, :].reshape(TM, 1)
    scale = jnp.where((rows >= start) & (rows < end), scale, 0.0)
    y_ref[...] += jnp.dot(h * scale, w2_ref[0],
                          preferred_element_type=jnp.float32)


def _grouped_gemm(xs, w1, w2, scale_tiles, meta, interpret=False):
    grid_spec = pltpu.PrefetchScalarGridSpec(
        num_scalar_prefetch=1,
        grid=(L, F),
        in_specs=[
            pl.BlockSpec((TM, HS), lambda l, f, meta: (meta[L + l], 0)),
            pl.BlockSpec((1, HS, FFT), lambda l, f, meta: (meta[l], 0, f)),
            pl.BlockSpec((1, FFT, HS), lambda l, f, meta: (meta[l], f, 0)),
            pl.BlockSpec((1, 1, TM), lambda l, f, meta: (meta[L + l], 0, 0)),
        ],
        out_specs=pl.BlockSpec((TM, HS), lambda l, f, meta: (meta[L + l], 0)),
    )
    return pl.pallas_call(
        _grouped_body,
        grid_spec=grid_spec,
        out_shape=jax.ShapeDtypeStruct((M, HS), jnp.float32),
        interpret=interpret,
    )(meta, xs, w1, w2, scale_tiles)


def _shared_body(x_ref, w1_ref, w2_ref, y_ref):
    f = pl.program_id(1)

    @pl.when(f == 0)
    def _():
        y_ref[...] = jnp.zeros_like(y_ref)

    h = jax.nn.gelu(jnp.dot(x_ref[...], w1_ref[...],
                            preferred_element_type=jnp.float32))
    y_ref[...] += jnp.dot(h, w2_ref[...], preferred_element_type=jnp.float32)


def _shared_mlp(xf, w1_s, w2_s, interpret=False):
    return pl.pallas_call(
        _shared_body,
        grid=(NTS, F),
        in_specs=[
            pl.BlockSpec((TMS, HS), lambda m, f: (m, 0)),
            pl.BlockSpec((HS, FFT), lambda m, f: (0, f)),
            pl.BlockSpec((FFT, HS), lambda m, f: (f, 0)),
        ],
        out_specs=pl.BlockSpec((TMS, HS), lambda m, f: (m, 0)),
        out_shape=jax.ShapeDtypeStruct((SL, HS), jnp.float32),
        interpret=interpret,
    )(xf, w1_s, w2_s)


def _tile_metadata(tpe, bins):
    starts = bins - tpe
    t0 = starts // TM
    t1 = (bins - 1) // TM
    n = jnp.where(tpe > 0, t1 - t0 + 1, 0)
    cum = jnp.cumsum(n)
    total = cum[-1]
    l = jnp.arange(L, dtype=jnp.int32)
    g = jnp.searchsorted(cum, l, side='right').astype(jnp.int32)
    gc = jnp.clip(g, 0, NUM_EXPERTS - 1)
    j = l - (cum[gc] - n[gc])
    valid = l < total
    m_l = jnp.where(valid, starts[gc] // TM + j, NT - 1)
    s_l = jnp.where(valid, starts[gc], 1)
    e_l = jnp.where(valid, bins[gc], 0)
    prev = jnp.concatenate([jnp.full((1,), -1, jnp.int32), m_l[:-1]])
    first = (m_l != prev).astype(jnp.int32)
    return jnp.concatenate([gc, m_l, s_l, e_l, first]).astype(jnp.int32)


def kernel(x, expert_weights, expert_indices, w1, w2, w1_s, w2_s):
    in_shape = x.shape
    xf = x.reshape(-1, HS)

    top = expert_indices.reshape(-1).astype(jnp.int32)
    sort_idx = jnp.argsort(top)
    pos_of_slot = jnp.argsort(sort_idx).astype(jnp.int32)
    sorted_tok = (sort_idx // TOP_K).astype(jnp.int32)
    w_sorted = expert_weights.reshape(-1)[sort_idx]
    tpe = jnp.bincount(top, length=NUM_EXPERTS).astype(jnp.int32)
    bins = jnp.cumsum(tpe).astype(jnp.int32)

    meta = _tile_metadata(tpe, bins)

    xs = xf[sorted_tok]

    scale_tiles = w_sorted.reshape(NT, 1, TM)
    y = _grouped_gemm(xs, w1, w2, scale_tiles, meta)
    s = _shared_mlp(xf, w1_s, w2_s)

    contrib = y[pos_of_slot].reshape(SL, TOP_K, HS).sum(axis=1)
    out = s + contrib
    return out.reshape(in_shape)
